# Initial kernel scaffold; baseline (speedup 1.0000x reference)
import functools

import jax
import jax.numpy as jnp
import numpy as np
from jax import lax
from jax.experimental import pallas as pl
from jax.experimental.pallas import tpu as pltpu
from jax.experimental.pallas import tpu_sc as plsc

B, T, D = 4, 4096, 128
T4 = T // 4
W = 12
NW_CAND = 26
KB = 256
NB = 512
NEG = -1e9

_TID = np.array(sorted(int(v) for v in np.linspace(0, T - 1, T4)), dtype=np.int32)
_E = (_TID - 4 * np.arange(T4, dtype=np.int32)).astype(np.int32)


def _select_kernel(e_ref, x0_ref, x1_ref, x2_ref, x3_ref, out_ref):
    b = pl.program_id(0)
    kb = pl.program_id(1)
    xc = (x0_ref, x1_ref, x2_ref, x3_ref)

    k_loc = lax.broadcasted_iota(jnp.int32, (KB, 1), 0)
    k_glob = kb * KB + k_loc
    e = e_ref[...]

    base_q = kb * KB
    sel_x = jnp.zeros((KB, D), jnp.float32)
    for c in range(4):
        seg = xc[c][0, base_q + 4:base_q + 4 + KB, :]
        sel_x = jnp.where(e == c, seg, sel_x)

    s_cols = []
    for w in range(NW_CAND):
        c = (w + 5) % 4
        q0 = base_q + (w + 5) // 4
        win = xc[c][0, q0:q0 + KB, :]
        sraw = jnp.sum(sel_x * win, axis=1, keepdims=True)
        j = 4 * k_glob + (w - 11)
        valid = (w >= e) & (w <= e + 22) & (j >= 0) & (j <= T - 1)
        s_cols.append(jnp.where(valid, sraw, NEG))
    S = jnp.concatenate(s_cols, axis=1)

    lane_w = lax.broadcasted_iota(jnp.int32, (KB, NW_CAND), 1)
    t_lane = lax.broadcasted_iota(jnp.int32, (KB, W), 1)
    running = jnp.zeros((KB, 1), jnp.int32)
    acc = jnp.zeros((KB, W), jnp.int32)
    for w in range(NW_CAND):
        m = S[:, w:w + 1]
        beats = (S > m) | ((S == m) & (lane_w < w))
        rank = jnp.sum(beats.astype(jnp.int32), axis=1, keepdims=True)
        keep = rank < W
        pos = running
        running = running + keep.astype(jnp.int32)
        jval = 4 * k_glob + (w - 11) + b * T
        acc = acc + jnp.where(keep & (pos == t_lane), jval, 0)
    out_ref[0] = acc


def _run_select(e2, xcs):
    grid = (B, T4 // KB)
    qn = xcs[0].shape[1]
    in_specs = [pl.BlockSpec((KB, 1), lambda b, kb: (kb, 0))] + [
        pl.BlockSpec((1, qn, D), lambda b, kb: (b, 0, 0))
        for _ in range(4)
    ]
    return pl.pallas_call(
        _select_kernel,
        grid=grid,
        in_specs=in_specs,
        out_specs=pl.BlockSpec((1, KB, W), lambda b, kb: (b, kb, 0)),
        out_shape=jax.ShapeDtypeStruct((B, T4, W), jnp.int32),
    )(e2, *xcs)


_SC_CHUNK = 128
_SC_INFO = plsc.get_sparse_core_info()
_SC_NW = _SC_INFO.num_cores * _SC_INFO.num_subcores


def _sc_gather_body(table_hbm, idx_hbm, out_hbm, idx_v, rows_v, sem):
    nrows = B * T4 * W
    per_w = nrows // _SC_NW
    nchunk = per_w // _SC_CHUNK
    wid = lax.axis_index("s") * _SC_INFO.num_cores + lax.axis_index("c")
    base = wid * per_w
    pltpu.sync_copy(idx_hbm.at[pl.ds(base, per_w)], idx_v)
    for ci in range(nchunk):
        idx_c = idx_v.at[pl.ds(ci * _SC_CHUNK, _SC_CHUNK)]
        pltpu.async_copy(table_hbm.at[idx_c], rows_v, sem).wait()
        pltpu.sync_copy(rows_v, out_hbm.at[pl.ds(base + ci * _SC_CHUNK, _SC_CHUNK)])


def _sc_gather(xflat, idxflat):
    nrows = B * T4 * W
    per_w = nrows // _SC_NW
    mesh = plsc.VectorSubcoreMesh(core_axis_name="c", subcore_axis_name="s")
    k = functools.partial(
        pl.kernel,
        mesh=mesh,
        out_type=jax.ShapeDtypeStruct((nrows, D), jnp.float32),
        scratch_types=[
            pltpu.VMEM((per_w,), jnp.int32),
            pltpu.VMEM((_SC_CHUNK, D), jnp.float32),
            pltpu.SemaphoreType.DMA,
        ],
    )(_sc_gather_body)
    return k(xflat, idxflat)


def _gru_kernel(f_ref, wih_ref, whh_ref, bih_ref, bhh_ref, out_ref):
    wih = wih_ref[...]
    whh = whh_ref[...]
    bih = bih_ref[...]
    bhh = bhh_ref[...]
    h = jnp.zeros((NB, D), jnp.float32)
    for t in range(W):
        xt = f_ref[t]
        gi = jnp.dot(xt, wih, preferred_element_type=jnp.float32) + bih
        gh = jnp.dot(h, whh, preferred_element_type=jnp.float32) + bhh
        r = jax.nn.sigmoid(gi[:, :D] + gh[:, :D])
        z = jax.nn.sigmoid(gi[:, D:2 * D] + gh[:, D:2 * D])
        n = jnp.tanh(gi[:, 2 * D:] + r * gh[:, 2 * D:])
        h = (1.0 - z) * n + z * h
    out_ref[...] = h


def _run_gru(feat, wihT, whhT, bih2, bhh2):
    ntot = B * T4
    grid = (ntot // NB,)
    return pl.pallas_call(
        _gru_kernel,
        grid=grid,
        in_specs=[
            pl.BlockSpec((W, NB, D), lambda nb: (0, nb, 0)),
            pl.BlockSpec((D, 3 * D), lambda nb: (0, 0)),
            pl.BlockSpec((D, 3 * D), lambda nb: (0, 0)),
            pl.BlockSpec((1, 3 * D), lambda nb: (0, 0)),
            pl.BlockSpec((1, 3 * D), lambda nb: (0, 0)),
        ],
        out_specs=pl.BlockSpec((NB, D), lambda nb: (nb, 0)),
        out_shape=jax.ShapeDtypeStruct((ntot, D), jnp.float32),
    )(feat, wihT, whhT, bih2, bhh2)


def kernel(x, W_ih, W_hh, b_ih, b_hh):
    xp = jnp.pad(x, ((0, 0), (16, 16), (0, 0)))
    xr = xp.reshape(B, (T + 32) // 4, 4, D)
    xcs = [xr[:, :, c, :] for c in range(4)]
    e2 = jnp.asarray(_E).reshape(T4, 1)

    idx = _run_select(e2, xcs)

    idx_t_major = jnp.transpose(idx, (2, 0, 1)).reshape(W * B * T4)
    xflat = x.reshape(B * T, D)
    feat = _sc_gather(xflat, idx_t_major)
    feat = feat.reshape(W, B * T4, D)

    h = _run_gru(feat, W_ih.T, W_hh.T, b_ih.reshape(1, 3 * D), b_hh.reshape(1, 3 * D))
    return h.reshape(B, T4, D)


# speedup vs baseline: 64.3590x; 64.3590x over previous
import functools

import jax
import jax.numpy as jnp
import numpy as np
from jax import lax
from jax.experimental import pallas as pl
from jax.experimental.pallas import tpu as pltpu
from jax.experimental.pallas import tpu_sc as plsc

B, T, D = 4, 4096, 128
T4 = T // 4
W = 12
NW_CAND = 26
KB = 256
NB = 512
NEG = -1e9

_TID = np.array(sorted(int(v) for v in np.linspace(0, T - 1, T4)), dtype=np.int32)
_E = (_TID - 4 * np.arange(T4, dtype=np.int32)).astype(np.int32)


NCOL = 4 * KB + 32


def _select_kernel(e_ref, xp_ref, x0_ref, x1_ref, x2_ref, x3_ref, out_ref):
    b = pl.program_id(0)
    kb = pl.program_id(1)
    xc = (x0_ref, x1_ref, x2_ref, x3_ref)

    k_loc = lax.broadcasted_iota(jnp.int32, (KB, 1), 0)
    k_glob = kb * KB + k_loc
    e = e_ref[...]

    base_q = kb * KB
    sel_x = jnp.zeros((KB, D), jnp.float32)
    for c in range(4):
        seg = xc[c][0, pl.ds(base_q + 4, KB), :]
        sel_x = jnp.where(e == c, seg, sel_x)

    cols = xp_ref[0, pl.ds(kb * (4 * KB), NCOL), :]
    S_blk = lax.dot_general(sel_x, cols, (((1,), (1,)), ((), ())))

    lane_c = lax.broadcasted_iota(jnp.int32, (KB, NCOL), 1)
    s_cols = []
    for w in range(NW_CAND):
        m = lane_c == (4 * k_loc + w + 5)
        sraw = jnp.sum(jnp.where(m, S_blk, 0.0), axis=1, keepdims=True)
        j = 4 * k_glob + (w - 11)
        valid = (w >= e) & (w <= e + 22) & (j >= 0) & (j <= T - 1)
        s_cols.append(jnp.where(valid, sraw, NEG))
    S = jnp.concatenate(s_cols, axis=1)

    lane_w = lax.broadcasted_iota(jnp.int32, (KB, NW_CAND), 1)
    t_lane = lax.broadcasted_iota(jnp.int32, (KB, W), 1)
    running = jnp.zeros((KB, 1), jnp.int32)
    acc = jnp.zeros((KB, W), jnp.int32)
    for w in range(NW_CAND):
        m = S[:, w:w + 1]
        beats = (S > m) | ((S == m) & (lane_w < w))
        rank = jnp.sum(beats.astype(jnp.int32), axis=1, keepdims=True)
        keep = rank < W
        pos = running
        running = running + keep.astype(jnp.int32)
        jval = 4 * k_glob + (w - 11) + b * T
        acc = acc + jnp.where(keep & (pos == t_lane), jval, 0)
    out_ref[0] = acc


def _run_select(e2, xp, xcs):
    grid = (B, T4 // KB)
    qn = xcs[0].shape[1]
    in_specs = [
        pl.BlockSpec((KB, 1), lambda b, kb: (kb, 0)),
        pl.BlockSpec((1, T + 32, D), lambda b, kb: (b, 0, 0)),
    ] + [
        pl.BlockSpec((1, qn, D), lambda b, kb: (b, 0, 0))
        for _ in range(4)
    ]
    return pl.pallas_call(
        _select_kernel,
        grid=grid,
        in_specs=in_specs,
        out_specs=pl.BlockSpec((1, KB, W), lambda b, kb: (b, kb, 0)),
        out_shape=jax.ShapeDtypeStruct((B, T4, W), jnp.int32),
    )(e2, xp, *xcs)


_SC_CHUNK = 128
_SC_NC = 2
_SC_NS = 16
_SC_NW = _SC_NC * _SC_NS


def _sc_gather_body(table_hbm, idx_hbm, out_hbm, idx_v, rows_v, sem):
    nrows = B * T4 * W
    per_w = nrows // _SC_NW
    nchunk = per_w // _SC_CHUNK
    wid = lax.axis_index("s") * _SC_NC + lax.axis_index("c")
    base = wid * per_w
    pltpu.sync_copy(idx_hbm.at[pl.ds(base, per_w)], idx_v)
    for ci in range(nchunk):
        idx_c = idx_v.at[pl.ds(ci * _SC_CHUNK, _SC_CHUNK)]
        pltpu.async_copy(table_hbm.at[idx_c], rows_v, sem).wait()
        pltpu.sync_copy(rows_v, out_hbm.at[pl.ds(base + ci * _SC_CHUNK, _SC_CHUNK)])


def _sc_gather(xflat, idxflat):
    nrows = B * T4 * W
    per_w = nrows // _SC_NW
    mesh = plsc.VectorSubcoreMesh(core_axis_name="c", subcore_axis_name="s")
    k = functools.partial(
        pl.kernel,
        mesh=mesh,
        out_type=jax.ShapeDtypeStruct((nrows, D), jnp.float32),
        scratch_types=[
            pltpu.VMEM((per_w,), jnp.int32),
            pltpu.VMEM((_SC_CHUNK, D), jnp.float32),
            pltpu.SemaphoreType.DMA,
        ],
    )(_sc_gather_body)
    return k(xflat, idxflat)


def _gru_kernel(f_ref, wih_ref, whh_ref, bih_ref, bhh_ref, out_ref):
    wih = wih_ref[...]
    whh = whh_ref[...]
    bih = bih_ref[...]
    bhh = bhh_ref[...]
    h = jnp.zeros((NB, D), jnp.float32)
    for t in range(W):
        xt = f_ref[t]
        gi = jnp.dot(xt, wih, preferred_element_type=jnp.float32) + bih
        gh = jnp.dot(h, whh, preferred_element_type=jnp.float32) + bhh
        r = jax.nn.sigmoid(gi[:, :D] + gh[:, :D])
        z = jax.nn.sigmoid(gi[:, D:2 * D] + gh[:, D:2 * D])
        n = jnp.tanh(gi[:, 2 * D:] + r * gh[:, 2 * D:])
        h = (1.0 - z) * n + z * h
    out_ref[...] = h


def _run_gru(feat, wihT, whhT, bih2, bhh2):
    ntot = B * T4
    grid = (ntot // NB,)
    return pl.pallas_call(
        _gru_kernel,
        grid=grid,
        in_specs=[
            pl.BlockSpec((W, NB, D), lambda nb: (0, nb, 0)),
            pl.BlockSpec((D, 3 * D), lambda nb: (0, 0)),
            pl.BlockSpec((D, 3 * D), lambda nb: (0, 0)),
            pl.BlockSpec((1, 3 * D), lambda nb: (0, 0)),
            pl.BlockSpec((1, 3 * D), lambda nb: (0, 0)),
        ],
        out_specs=pl.BlockSpec((NB, D), lambda nb: (nb, 0)),
        out_shape=jax.ShapeDtypeStruct((ntot, D), jnp.float32),
    )(feat, wihT, whhT, bih2, bhh2)


def kernel(x, W_ih, W_hh, b_ih, b_hh):
    xp = jnp.pad(x, ((0, 0), (16, 16), (0, 0)))
    xr = xp.reshape(B, (T + 32) // 4, 4, D)
    xcs = [xr[:, :, c, :] for c in range(4)]
    e2 = jnp.asarray(_E).reshape(T4, 1)

    idx = _run_select(e2, xp, xcs)

    idx_t_major = jnp.transpose(idx, (2, 0, 1)).reshape(W * B * T4)
    xflat = x.reshape(B * T, D)
    feat = _sc_gather(xflat, idx_t_major)
    feat = feat.reshape(W, B * T4, D)

    h = _run_gru(feat, W_ih.T, W_hh.T, b_ih.reshape(1, 3 * D), b_hh.reshape(1, 3 * D))
    return h.reshape(B, T4, D)


# speedup vs baseline: 70.3100x; 1.0925x over previous
import functools

import jax
import jax.numpy as jnp
import numpy as np
from jax import lax
from jax.experimental import pallas as pl
from jax.experimental.pallas import tpu as pltpu
from jax.experimental.pallas import tpu_sc as plsc

B, T, D = 4, 4096, 128
T4 = T // 4
W = 12
NW_CAND = 26
KB = 256
NB = 512
NEG = -1e9

_TID = np.array(sorted(int(v) for v in np.linspace(0, T - 1, T4)), dtype=np.int32)
_E = (_TID - 4 * np.arange(T4, dtype=np.int32)).astype(np.int32)


NCOL = 4 * KB + 32


def _select_kernel(e_ref, xp_ref, x0_ref, x1_ref, x2_ref, x3_ref, out_ref):
    b = pl.program_id(0)
    kb = pl.program_id(1)
    xc = (x0_ref, x1_ref, x2_ref, x3_ref)

    k_loc = lax.broadcasted_iota(jnp.int32, (KB, 1), 0)
    k_glob = kb * KB + k_loc
    e = e_ref[...]

    base_q = kb * KB
    sel_x = jnp.zeros((KB, D), jnp.float32)
    for c in range(4):
        seg = xc[c][0, pl.ds(base_q + 4, KB), :]
        sel_x = jnp.where(e == c, seg, sel_x)

    KS = 32
    NCS = 4 * KS + 32
    lane_c = lax.broadcasted_iota(jnp.int32, (KS, NCS), 1)
    kl = lax.broadcasted_iota(jnp.int32, (KS, 1), 0)
    tgt0 = 4 * kl + 5
    s_sub_rows = []
    for ks in range(0, KB, KS):
        sel_sub = sel_x[ks:ks + KS, :]
        cols = xp_ref[0, pl.ds(kb * (4 * KB) + 4 * ks, NCS), :]
        S_sub = lax.dot_general(sel_sub, cols, (((1,), (1,)), ((), ())))
        sw = []
        for w in range(NW_CAND):
            m = lane_c == (tgt0 + w)
            sw.append(jnp.sum(jnp.where(m, S_sub, 0.0), axis=1, keepdims=True))
        s_sub_rows.append(jnp.concatenate(sw, axis=1))
    Sraw = jnp.concatenate(s_sub_rows, axis=0)

    s_cols = []
    for w in range(NW_CAND):
        j = 4 * k_glob + (w - 11)
        valid = (w >= e) & (w <= e + 22) & (j >= 0) & (j <= T - 1)
        s_cols.append(jnp.where(valid, Sraw[:, w:w + 1], NEG))
    S = jnp.concatenate(s_cols, axis=1)

    lane_w = lax.broadcasted_iota(jnp.int32, (KB, NW_CAND), 1)
    t_lane = lax.broadcasted_iota(jnp.int32, (KB, W), 1)
    running = jnp.zeros((KB, 1), jnp.int32)
    acc = jnp.zeros((KB, W), jnp.int32)
    for w in range(NW_CAND):
        m = S[:, w:w + 1]
        beats = (S > m) | ((S == m) & (lane_w < w))
        rank = jnp.sum(beats.astype(jnp.int32), axis=1, keepdims=True)
        keep = rank < W
        pos = running
        running = running + keep.astype(jnp.int32)
        jval = 4 * k_glob + (w - 11) + b * T
        acc = acc + jnp.where(keep & (pos == t_lane), jval, 0)
    out_ref[0] = acc


def _run_select(e2, xp, xcs):
    grid = (B, T4 // KB)
    qn = xcs[0].shape[1]
    in_specs = [
        pl.BlockSpec((KB, 1), lambda b, kb: (kb, 0)),
        pl.BlockSpec((1, T + 32, D), lambda b, kb: (b, 0, 0)),
    ] + [
        pl.BlockSpec((1, qn, D), lambda b, kb: (b, 0, 0))
        for _ in range(4)
    ]
    return pl.pallas_call(
        _select_kernel,
        grid=grid,
        in_specs=in_specs,
        out_specs=pl.BlockSpec((1, KB, W), lambda b, kb: (b, kb, 0)),
        out_shape=jax.ShapeDtypeStruct((B, T4, W), jnp.int32),
    )(e2, xp, *xcs)


_SC_CHUNK = 128
_SC_NC = 2
_SC_NS = 16
_SC_NW = _SC_NC * _SC_NS


def _sc_gather_body(table_hbm, idx_hbm, out_hbm, idx_v, rows_v, sem):
    nrows = B * T4 * W
    per_w = nrows // _SC_NW
    nchunk = per_w // _SC_CHUNK
    wid = lax.axis_index("s") * _SC_NC + lax.axis_index("c")
    base = wid * per_w
    pltpu.sync_copy(idx_hbm.at[pl.ds(base, per_w)], idx_v)
    for ci in range(nchunk):
        idx_c = idx_v.at[pl.ds(ci * _SC_CHUNK, _SC_CHUNK)]
        pltpu.async_copy(table_hbm.at[idx_c], rows_v, sem).wait()
        pltpu.sync_copy(rows_v, out_hbm.at[pl.ds(base + ci * _SC_CHUNK, _SC_CHUNK)])


def _sc_gather(xflat, idxflat):
    nrows = B * T4 * W
    per_w = nrows // _SC_NW
    mesh = plsc.VectorSubcoreMesh(core_axis_name="c", subcore_axis_name="s")
    k = functools.partial(
        pl.kernel,
        mesh=mesh,
        out_type=jax.ShapeDtypeStruct((nrows, D), jnp.float32),
        scratch_types=[
            pltpu.VMEM((per_w,), jnp.int32),
            pltpu.VMEM((_SC_CHUNK, D), jnp.float32),
            pltpu.SemaphoreType.DMA,
        ],
    )(_sc_gather_body)
    return k(xflat, idxflat)


def _gru_kernel(f_ref, wih_ref, whh_ref, bih_ref, bhh_ref, out_ref):
    wih = wih_ref[...]
    whh = whh_ref[...]
    bih = bih_ref[...]
    bhh = bhh_ref[...]
    h = jnp.zeros((NB, D), jnp.float32)
    for t in range(W):
        xt = f_ref[t]
        gi = jnp.dot(xt, wih, preferred_element_type=jnp.float32) + bih
        gh = jnp.dot(h, whh, preferred_element_type=jnp.float32) + bhh
        r = jax.nn.sigmoid(gi[:, :D] + gh[:, :D])
        z = jax.nn.sigmoid(gi[:, D:2 * D] + gh[:, D:2 * D])
        n = jnp.tanh(gi[:, 2 * D:] + r * gh[:, 2 * D:])
        h = (1.0 - z) * n + z * h
    out_ref[...] = h


def _run_gru(feat, wihT, whhT, bih2, bhh2):
    ntot = B * T4
    grid = (ntot // NB,)
    return pl.pallas_call(
        _gru_kernel,
        grid=grid,
        in_specs=[
            pl.BlockSpec((W, NB, D), lambda nb: (0, nb, 0)),
            pl.BlockSpec((D, 3 * D), lambda nb: (0, 0)),
            pl.BlockSpec((D, 3 * D), lambda nb: (0, 0)),
            pl.BlockSpec((1, 3 * D), lambda nb: (0, 0)),
            pl.BlockSpec((1, 3 * D), lambda nb: (0, 0)),
        ],
        out_specs=pl.BlockSpec((NB, D), lambda nb: (nb, 0)),
        out_shape=jax.ShapeDtypeStruct((ntot, D), jnp.float32),
    )(feat, wihT, whhT, bih2, bhh2)


def kernel(x, W_ih, W_hh, b_ih, b_hh):
    xp = jnp.pad(x, ((0, 0), (16, 16), (0, 0)))
    xr = xp.reshape(B, (T + 32) // 4, 4, D)
    xcs = [xr[:, :, c, :] for c in range(4)]
    e2 = jnp.asarray(_E).reshape(T4, 1)

    idx = _run_select(e2, xp, xcs)

    idx_t_major = jnp.transpose(idx, (2, 0, 1)).reshape(W * B * T4)
    xflat = x.reshape(B * T, D)
    feat = _sc_gather(xflat, idx_t_major)
    feat = feat.reshape(W, B * T4, D)

    h = _run_gru(feat, W_ih.T, W_hh.T, b_ih.reshape(1, 3 * D), b_hh.reshape(1, 3 * D))
    return h.reshape(B, T4, D)


# speedup vs baseline: 129.4481x; 1.8411x over previous
import functools

import jax
import jax.numpy as jnp
import numpy as np
from jax import lax
from jax.experimental import pallas as pl
from jax.experimental.pallas import tpu as pltpu
from jax.experimental.pallas import tpu_sc as plsc

B, T, D = 4, 4096, 128
T4 = T // 4
W = 12
NW_CAND = 26
KB = 256
NB = 512
NEG = -1e9

_TID = np.array(sorted(int(v) for v in np.linspace(0, T - 1, T4)), dtype=np.int32)
_E = (_TID - 4 * np.arange(T4, dtype=np.int32)).astype(np.int32)


NCOL = 4 * KB + 32


def _select_kernel(e_ref, et_ref, xp_ref, x0_ref, x1_ref, x2_ref, x3_ref, out_ref):
    b = pl.program_id(0)
    kb = pl.program_id(1)
    xc = (x0_ref, x1_ref, x2_ref, x3_ref)

    e = e_ref[...]

    base_q = kb * KB
    sel_x = jnp.zeros((KB, D), jnp.float32)
    for c in range(4):
        seg = xc[c][0, pl.ds(base_q + 4, KB), :]
        sel_x = jnp.where(e == c, seg, sel_x)

    KS = 32
    NCS = 4 * KS + 32
    lane_c = lax.broadcasted_iota(jnp.int32, (KS, NCS), 1)
    kl = lax.broadcasted_iota(jnp.int32, (KS, 1), 0)
    tgt0 = 4 * kl + 5
    s_sub_rows = []
    for ks in range(0, KB, KS):
        sel_sub = sel_x[ks:ks + KS, :]
        cols = xp_ref[0, pl.ds(kb * (4 * KB) + 4 * ks, NCS), :]
        S_sub = lax.dot_general(sel_sub, cols, (((1,), (1,)), ((), ())))
        sw = []
        for w in range(NW_CAND):
            m = lane_c == (tgt0 + w)
            sw.append(jnp.sum(jnp.where(m, S_sub, 0.0), axis=1, keepdims=True))
        s_sub_rows.append(jnp.concatenate(sw, axis=1))
    Sraw = jnp.concatenate(s_sub_rows, axis=0)

    St = jnp.transpose(Sraw)
    eT = et_ref[...]
    k_lane = lax.broadcasted_iota(jnp.int32, (NW_CAND, KB), 1) + kb * KB
    w_sub = lax.broadcasted_iota(jnp.int32, (NW_CAND, KB), 0)
    jmat = 4 * k_lane + w_sub - 11
    valid = (w_sub >= eT) & (w_sub <= eT + 22) & (jmat >= 0) & (jmat <= T - 1)
    S = jnp.where(valid, St, NEG)

    Sp = S[:, None, :]
    Sq = S[None, :, :]
    wp = lax.broadcasted_iota(jnp.int32, (NW_CAND, NW_CAND, 1), 0)
    wq = lax.broadcasted_iota(jnp.int32, (NW_CAND, NW_CAND, 1), 1)
    beats = (Sp > Sq) | ((Sp == Sq) & (wp < wq))
    rank = jnp.sum(beats.astype(jnp.int32), axis=0)
    keep = rank < W
    pos = jnp.sum((keep[:, None, :] & (wp < wq)).astype(jnp.int32), axis=0)

    t_sub = lax.broadcasted_iota(jnp.int32, (NW_CAND, W, 1), 1)
    onehot = keep[:, None, :] & (pos[:, None, :] == t_sub)
    jabs = jmat + b * T
    acc = jnp.sum(jnp.where(onehot, jabs[:, None, :], 0), axis=0)
    out_ref[0] = acc


def _run_select(e2, et2, xp, xcs):
    grid = (B, T4 // KB)
    qn = xcs[0].shape[1]
    in_specs = [
        pl.BlockSpec((KB, 1), lambda b, kb: (kb, 0)),
        pl.BlockSpec((1, KB), lambda b, kb: (0, kb)),
        pl.BlockSpec((1, T + 32, D), lambda b, kb: (b, 0, 0)),
    ] + [
        pl.BlockSpec((1, qn, D), lambda b, kb: (b, 0, 0))
        for _ in range(4)
    ]
    return pl.pallas_call(
        _select_kernel,
        grid=grid,
        in_specs=in_specs,
        out_specs=pl.BlockSpec((1, W, KB), lambda b, kb: (b, 0, kb)),
        out_shape=jax.ShapeDtypeStruct((B, W, T4), jnp.int32),
    )(e2, et2, xp, *xcs)


_SC_CHUNK = 128
_SC_NC = 2
_SC_NS = 16
_SC_NW = _SC_NC * _SC_NS


def _sc_gather_body(table_hbm, idx_hbm, out_hbm, idx_v, rows_v, sem):
    nrows = B * T4 * W
    per_w = nrows // _SC_NW
    nchunk = per_w // _SC_CHUNK
    wid = lax.axis_index("s") * _SC_NC + lax.axis_index("c")
    base = wid * per_w
    pltpu.sync_copy(idx_hbm.at[pl.ds(base, per_w)], idx_v)
    for ci in range(nchunk):
        idx_c = idx_v.at[pl.ds(ci * _SC_CHUNK, _SC_CHUNK)]
        pltpu.async_copy(table_hbm.at[idx_c], rows_v, sem).wait()
        pltpu.sync_copy(rows_v, out_hbm.at[pl.ds(base + ci * _SC_CHUNK, _SC_CHUNK)])


def _sc_gather(xflat, idxflat):
    nrows = B * T4 * W
    per_w = nrows // _SC_NW
    mesh = plsc.VectorSubcoreMesh(core_axis_name="c", subcore_axis_name="s")
    k = functools.partial(
        pl.kernel,
        mesh=mesh,
        out_type=jax.ShapeDtypeStruct((nrows, D), jnp.float32),
        scratch_types=[
            pltpu.VMEM((per_w,), jnp.int32),
            pltpu.VMEM((_SC_CHUNK, D), jnp.float32),
            pltpu.SemaphoreType.DMA,
        ],
    )(_sc_gather_body)
    return k(xflat, idxflat)


def _gru_kernel(f_ref, wih_ref, whh_ref, bih_ref, bhh_ref, out_ref):
    wih = wih_ref[...]
    whh = whh_ref[...]
    bih = bih_ref[...]
    bhh = bhh_ref[...]
    h = jnp.zeros((NB, D), jnp.float32)
    for t in range(W):
        xt = f_ref[t]
        gi = jnp.dot(xt, wih, preferred_element_type=jnp.float32) + bih
        gh = jnp.dot(h, whh, preferred_element_type=jnp.float32) + bhh
        r = jax.nn.sigmoid(gi[:, :D] + gh[:, :D])
        z = jax.nn.sigmoid(gi[:, D:2 * D] + gh[:, D:2 * D])
        n = jnp.tanh(gi[:, 2 * D:] + r * gh[:, 2 * D:])
        h = (1.0 - z) * n + z * h
    out_ref[...] = h


def _run_gru(feat, wihT, whhT, bih2, bhh2):
    ntot = B * T4
    grid = (ntot // NB,)
    return pl.pallas_call(
        _gru_kernel,
        grid=grid,
        in_specs=[
            pl.BlockSpec((W, NB, D), lambda nb: (0, nb, 0)),
            pl.BlockSpec((D, 3 * D), lambda nb: (0, 0)),
            pl.BlockSpec((D, 3 * D), lambda nb: (0, 0)),
            pl.BlockSpec((1, 3 * D), lambda nb: (0, 0)),
            pl.BlockSpec((1, 3 * D), lambda nb: (0, 0)),
        ],
        out_specs=pl.BlockSpec((NB, D), lambda nb: (nb, 0)),
        out_shape=jax.ShapeDtypeStruct((ntot, D), jnp.float32),
    )(feat, wihT, whhT, bih2, bhh2)


def kernel(x, W_ih, W_hh, b_ih, b_hh):
    xp = jnp.pad(x, ((0, 0), (16, 16), (0, 0)))
    xr = xp.reshape(B, (T + 32) // 4, 4, D)
    xcs = [xr[:, :, c, :] for c in range(4)]
    e2 = jnp.asarray(_E).reshape(T4, 1)
    et2 = jnp.asarray(_E).reshape(1, T4)

    idx = _run_select(e2, et2, xp, xcs)

    idx_t_major = jnp.transpose(idx, (1, 0, 2)).reshape(W * B * T4)
    xflat = x.reshape(B * T, D)
    feat = _sc_gather(xflat, idx_t_major)
    feat = feat.reshape(W, B * T4, D)

    h = _run_gru(feat, W_ih.T, W_hh.T, b_ih.reshape(1, 3 * D), b_hh.reshape(1, 3 * D))
    return h.reshape(B, T4, D)


# speedup vs baseline: 133.9280x; 1.0346x over previous
import functools

import jax
import jax.numpy as jnp
import numpy as np
from jax import lax
from jax.experimental import pallas as pl
from jax.experimental.pallas import tpu as pltpu
from jax.experimental.pallas import tpu_sc as plsc

B, T, D = 4, 4096, 128
T4 = T // 4
W = 12
NW_CAND = 26
KB = 256
NB = 1024
NEG = -1e9

_TID = np.array(sorted(int(v) for v in np.linspace(0, T - 1, T4)), dtype=np.int32)
_E = (_TID - 4 * np.arange(T4, dtype=np.int32)).astype(np.int32)


NCOL = 4 * KB + 32


def _select_kernel(e_ref, et_ref, xp_ref, x0_ref, x1_ref, x2_ref, x3_ref, out_ref):
    b = pl.program_id(0)
    kb = pl.program_id(1)
    xc = (x0_ref, x1_ref, x2_ref, x3_ref)

    e = e_ref[...]

    base_q = kb * KB
    sel_x = jnp.zeros((KB, D), jnp.float32)
    for c in range(4):
        seg = xc[c][0, pl.ds(base_q + 4, KB), :]
        sel_x = jnp.where(e == c, seg, sel_x)

    KS = 32
    NCS = 4 * KS + 32
    lane_c = lax.broadcasted_iota(jnp.int32, (KS, NCS), 1)
    kl = lax.broadcasted_iota(jnp.int32, (KS, 1), 0)
    tgt0 = 4 * kl + 5
    s_sub_rows = []
    for ks in range(0, KB, KS):
        sel_sub = sel_x[ks:ks + KS, :]
        cols = xp_ref[0, pl.ds(kb * (4 * KB) + 4 * ks, NCS), :]
        S_sub = lax.dot_general(sel_sub, cols, (((1,), (1,)), ((), ())))
        sw = []
        for w in range(NW_CAND):
            m = lane_c == (tgt0 + w)
            sw.append(jnp.sum(jnp.where(m, S_sub, 0.0), axis=1, keepdims=True))
        s_sub_rows.append(jnp.concatenate(sw, axis=1))
    Sraw = jnp.concatenate(s_sub_rows, axis=0)

    St = jnp.transpose(Sraw)
    eT = et_ref[...]
    k_lane = lax.broadcasted_iota(jnp.int32, (NW_CAND, KB), 1) + kb * KB
    w_sub = lax.broadcasted_iota(jnp.int32, (NW_CAND, KB), 0)
    jmat = 4 * k_lane + w_sub - 11
    valid = (w_sub >= eT) & (w_sub <= eT + 22) & (jmat >= 0) & (jmat <= T - 1)
    S = jnp.where(valid, St, NEG)

    Sp = S[:, None, :]
    Sq = S[None, :, :]
    wp = lax.broadcasted_iota(jnp.int32, (NW_CAND, NW_CAND, 1), 0)
    wq = lax.broadcasted_iota(jnp.int32, (NW_CAND, NW_CAND, 1), 1)
    beats = (Sp > Sq) | ((Sp == Sq) & (wp < wq))
    rank = jnp.sum(beats.astype(jnp.int32), axis=0)
    keep = rank < W
    pos = jnp.sum((keep[:, None, :] & (wp < wq)).astype(jnp.int32), axis=0)

    t_sub = lax.broadcasted_iota(jnp.int32, (NW_CAND, W, 1), 1)
    onehot = keep[:, None, :] & (pos[:, None, :] == t_sub)
    jabs = jmat + b * T
    acc = jnp.sum(jnp.where(onehot, jabs[:, None, :], 0), axis=0)
    out_ref[...] = acc


def _run_select(e2, et2, xp, xcs):
    grid = (B, T4 // KB)
    qn = xcs[0].shape[1]
    in_specs = [
        pl.BlockSpec((KB, 1), lambda b, kb: (kb, 0)),
        pl.BlockSpec((1, KB), lambda b, kb: (0, kb)),
        pl.BlockSpec((1, T + 32, D), lambda b, kb: (b, 0, 0)),
    ] + [
        pl.BlockSpec((1, qn, D), lambda b, kb: (b, 0, 0))
        for _ in range(4)
    ]
    return pl.pallas_call(
        _select_kernel,
        grid=grid,
        in_specs=in_specs,
        out_specs=pl.BlockSpec((W, KB), lambda b, kb: (0, b * (T4 // KB) + kb)),
        out_shape=jax.ShapeDtypeStruct((W, B * T4), jnp.int32),
    )(e2, et2, xp, *xcs)


_SC_CHUNK = 128
_SC_NC = 2
_SC_NS = 16
_SC_NW = _SC_NC * _SC_NS


def _sc_gather_body(table_hbm, idx_hbm, out_hbm, idx_v, r0, r1, gs0, gs1, ss0, ss1):
    nrows = B * T4 * W
    per_w = nrows // _SC_NW
    nchunk = per_w // _SC_CHUNK
    wid = lax.axis_index("s") * _SC_NC + lax.axis_index("c")
    base = wid * per_w
    bufs, gsems, ssems = (r0, r1), (gs0, gs1), (ss0, ss1)
    scat = [None, None]
    pltpu.sync_copy(idx_hbm.at[pl.ds(base, per_w)], idx_v)
    for ci in range(nchunk):
        s = ci & 1
        if scat[s] is not None:
            scat[s].wait()
        idx_c = idx_v.at[pl.ds(ci * _SC_CHUNK, _SC_CHUNK)]
        pltpu.async_copy(table_hbm.at[idx_c], bufs[s], gsems[s]).wait()
        scat[s] = pltpu.async_copy(
            bufs[s], out_hbm.at[pl.ds(base + ci * _SC_CHUNK, _SC_CHUNK)], ssems[s])
    scat[0].wait()
    scat[1].wait()


def _sc_gather(xflat, idxflat):
    nrows = B * T4 * W
    per_w = nrows // _SC_NW
    mesh = plsc.VectorSubcoreMesh(core_axis_name="c", subcore_axis_name="s")
    k = functools.partial(
        pl.kernel,
        mesh=mesh,
        out_type=jax.ShapeDtypeStruct((nrows, D), jnp.float32),
        scratch_types=[
            pltpu.VMEM((per_w,), jnp.int32),
            pltpu.VMEM((_SC_CHUNK, D), jnp.float32),
            pltpu.VMEM((_SC_CHUNK, D), jnp.float32),
            pltpu.SemaphoreType.DMA,
            pltpu.SemaphoreType.DMA,
            pltpu.SemaphoreType.DMA,
            pltpu.SemaphoreType.DMA,
        ],
    )(_sc_gather_body)
    return k(xflat, idxflat)


def _gru_kernel(f_ref, wih_ref, whh_ref, bih_ref, bhh_ref, out_ref):
    wih = wih_ref[...]
    whh = whh_ref[...]
    bih = bih_ref[...]
    bhh = bhh_ref[...]
    h = jnp.zeros((NB, D), jnp.float32)
    for t in range(W):
        xt = f_ref[t]
        gi = jnp.dot(xt, wih, preferred_element_type=jnp.float32) + bih
        gh = jnp.dot(h, whh, preferred_element_type=jnp.float32) + bhh
        r = jax.nn.sigmoid(gi[:, :D] + gh[:, :D])
        z = jax.nn.sigmoid(gi[:, D:2 * D] + gh[:, D:2 * D])
        n = jnp.tanh(gi[:, 2 * D:] + r * gh[:, 2 * D:])
        h = (1.0 - z) * n + z * h
    out_ref[...] = h


def _run_gru(feat, wihT, whhT, bih2, bhh2):
    ntot = B * T4
    grid = (ntot // NB,)
    return pl.pallas_call(
        _gru_kernel,
        grid=grid,
        in_specs=[
            pl.BlockSpec((W, NB, D), lambda nb: (0, nb, 0)),
            pl.BlockSpec((D, 3 * D), lambda nb: (0, 0)),
            pl.BlockSpec((D, 3 * D), lambda nb: (0, 0)),
            pl.BlockSpec((1, 3 * D), lambda nb: (0, 0)),
            pl.BlockSpec((1, 3 * D), lambda nb: (0, 0)),
        ],
        out_specs=pl.BlockSpec((NB, D), lambda nb: (nb, 0)),
        out_shape=jax.ShapeDtypeStruct((ntot, D), jnp.float32),
    )(feat, wihT, whhT, bih2, bhh2)


def kernel(x, W_ih, W_hh, b_ih, b_hh):
    xp = jnp.pad(x, ((0, 0), (16, 16), (0, 0)))
    xr = xp.reshape(B, (T + 32) // 4, 4, D)
    xcs = [xr[:, :, c, :] for c in range(4)]
    e2 = jnp.asarray(_E).reshape(T4, 1)
    et2 = jnp.asarray(_E).reshape(1, T4)

    idx = _run_select(e2, et2, xp, xcs)

    idx_t_major = idx.reshape(W * B * T4)
    xflat = x.reshape(B * T, D)
    feat = _sc_gather(xflat, idx_t_major)
    feat = feat.reshape(W, B * T4, D)

    h = _run_gru(feat, W_ih.T, W_hh.T, b_ih.reshape(1, 3 * D), b_hh.reshape(1, 3 * D))
    return h.reshape(B, T4, D)
